# baseline (speedup 1.0000x reference)
import functools

import jax
import jax.numpy as jnp
from jax import lax
from jax.experimental import pallas as pl
from jax.experimental.pallas import tpu as pltpu
from jax.experimental.pallas import tpu_sc as plsc


def _row_gather(src, idx, P):
    R, D = src.shape
    info = plsc.get_sparse_core_info()
    NW = info.num_cores * info.num_subcores
    rpw = P // NW
    C = 16
    while rpw % C:
        C //= 2
    nch = rpw // C
    mesh = plsc.VectorSubcoreMesh(core_axis_name="c", subcore_axis_name="s")

    @functools.partial(
        pl.kernel, mesh=mesh,
        out_type=jax.ShapeDtypeStruct((P, D), src.dtype),
        scratch_types=[
            pltpu.VMEM((rpw,), jnp.int32),
            pltpu.VMEM((C, D), src.dtype),
            pltpu.VMEM((C, D), src.dtype),
            pltpu.SemaphoreType.DMA,
            pltpu.SemaphoreType.DMA,
        ])
    def gk(src_hbm, idx_hbm, out_hbm, idx_v, buf0, buf1, sem0, sem1):
        wid = lax.axis_index("s") * info.num_cores + lax.axis_index("c")
        base = wid * rpw
        pltpu.sync_copy(idx_hbm.at[pl.ds(base, rpw)], idx_v)
        bufs = (buf0, buf1)
        sems = (sem0, sem1)
        pltpu.async_copy(src_hbm.at[idx_v.at[pl.ds(0, C)]], buf0, sem0)
        for i in range(nch):
            b, s = bufs[i % 2], sems[i % 2]
            pltpu.make_async_copy(
                src_hbm.at[idx_v.at[pl.ds(i * C, C)]], b, s).wait()
            if i + 1 < nch:
                pltpu.async_copy(
                    src_hbm.at[idx_v.at[pl.ds((i + 1) * C, C)]],
                    bufs[(i + 1) % 2], sems[(i + 1) % 2])
            pltpu.sync_copy(b, out_hbm.at[pl.ds(base + i * C, C)])

    return gk(src, idx)


def _row_scatter(src, idx, PO):
    P, D = src.shape
    info = plsc.get_sparse_core_info()
    NW = info.num_cores * info.num_subcores
    rpw = P // NW
    C = 16
    while rpw % C:
        C //= 2
    nch = rpw // C
    idx_r = idx.reshape(NW, nch, C)
    mesh = plsc.VectorSubcoreMesh(core_axis_name="c", subcore_axis_name="s")

    @functools.partial(
        pl.kernel, mesh=mesh,
        out_type=jax.ShapeDtypeStruct((PO, D), src.dtype),
        scratch_types=[
            pltpu.VMEM((nch, C), jnp.int32),
            pltpu.VMEM((C, D), src.dtype),
            pltpu.VMEM((C, D), src.dtype),
            pltpu.SemaphoreType.DMA,
            pltpu.SemaphoreType.DMA,
            pltpu.SemaphoreType.DMA,
        ])
    def sk(src_hbm, idx_hbm, out_hbm, idx_v, buf0, buf1, lsem0, lsem1, ssem):
        wid = lax.axis_index("s") * info.num_cores + lax.axis_index("c")
        base = wid * rpw
        pltpu.sync_copy(idx_hbm.at[wid], idx_v)
        bufs = (buf0, buf1)
        sems = (lsem0, lsem1)
        pltpu.async_copy(src_hbm.at[pl.ds(base, C)], buf0, lsem0)
        for i in range(nch):
            b, s = bufs[i % 2], sems[i % 2]
            pltpu.make_async_copy(
                src_hbm.at[pl.ds(base + i * C, C)], b, s).wait()
            if i + 1 < nch:
                pltpu.async_copy(
                    src_hbm.at[pl.ds(base + (i + 1) * C, C)],
                    bufs[(i + 1) % 2], sems[(i + 1) % 2])
            pltpu.async_copy(b, out_hbm.at[idx_v.at[i]], ssem)
            pltpu.make_async_copy(b, out_hbm.at[idx_v.at[i]], ssem).wait()

    return sk(src, idx_r)


def _dot_t(a, b):
    return jax.lax.dot_general(a, b, (((1,), (1,)), ((), ())),
                               preferred_element_type=jnp.float32)


def _dot_t_bf(a, b):
    return jax.lax.dot_general(a.astype(jnp.bfloat16), b.astype(jnp.bfloat16),
                               (((1,), (1,)), ((), ())),
                               preferred_element_type=jnp.float32)


def _router_body(x_ref, gw_ref, idx_ref, w_ref):
    x = x_ref[...]
    gw = gw_ref[...]
    logits = _dot_t(x, gw)
    e_num = logits.shape[1]
    iota = jax.lax.broadcasted_iota(jnp.int32, logits.shape, 1)
    m1 = jnp.max(logits, axis=1, keepdims=True)
    cand1 = jnp.where(logits == m1, iota, e_num)
    i1 = jnp.min(cand1, axis=1, keepdims=True)
    mask1 = iota == i1
    l2 = jnp.where(mask1, -jnp.inf, logits)
    m2 = jnp.max(l2, axis=1, keepdims=True)
    cand2 = jnp.where(l2 == m2, iota, e_num)
    i2 = jnp.min(cand2, axis=1, keepdims=True)
    b = jnp.exp(m2 - m1)
    w1 = 1.0 / (1.0 + b)
    w2 = 1.0 - w1
    idx_ref[...] = jnp.concatenate([i1, i2], axis=1)
    w_ref[...] = jnp.concatenate([w1, w2], axis=1)


def _up_body(te_ref, x_ref, wg_ref, wu_ref, h_ref):
    x = x_ref[...]
    g = _dot_t_bf(x, wg_ref[0])
    u = _dot_t_bf(x, wu_ref[0])
    h_ref[...] = (g * jax.nn.sigmoid(g)) * u


def _down_body(te_ref, h_ref, wd_ref, ws_ref, out_ref):
    h = h_ref[...]
    d = _dot_t_bf(h, wd_ref[0])
    out_ref[...] = ws_ref[0] * d


def _final_body(x_ref, sg_ref, su_ref, sd_ref, rp_ref, out_ref):
    f = pl.program_id(1)
    x = x_ref[...]
    g = _dot_t_bf(x, sg_ref[...])
    u = _dot_t_bf(x, su_ref[...])
    h = (g * jax.nn.sigmoid(g)) * u
    d = _dot_t_bf(h, sd_ref[...])

    @pl.when(f == 0)
    def _():
        rp = rp_ref[...]
        dd = d.shape[1]
        out_ref[...] = rp[:, :dd] + rp[:, dd:] + d

    @pl.when(f != 0)
    def _():
        out_ref[...] += d


def kernel(x, gate_w, w_up, w_down, sw_gate, sw_up, sw_down):
    B, T, D = x.shape
    N = B * T
    E = gate_w.shape[0]
    half = w_down.shape[2]
    xf = x.reshape(N, D)

    TB = 512 if N % 512 == 0 else N
    NT = N // TB
    TILE = 256 if N >= 2048 else 64
    NTT = (2 * N) // TILE + E
    P = NTT * TILE
    NF = 2 if half % 256 == 0 else 1
    F = half // NF

    topk_idx, topk_w = pl.pallas_call(
        _router_body,
        grid=(NT,),
        in_specs=[
            pl.BlockSpec((TB, D), lambda t: (t, 0)),
            pl.BlockSpec((E, D), lambda t: (0, 0)),
        ],
        out_specs=[
            pl.BlockSpec((TB, 2), lambda t: (t, 0)),
            pl.BlockSpec((TB, 2), lambda t: (t, 0)),
        ],
        out_shape=[
            jax.ShapeDtypeStruct((N, 2), jnp.int32),
            jax.ShapeDtypeStruct((N, 2), jnp.float32),
        ],
    )(xf, gate_w)

    i32 = jnp.int32
    ids = topk_idx.reshape(-1)
    wfl = topk_w.reshape(-1)
    order = jnp.argsort(ids, stable=True)
    sorted_ids = ids[order]
    counts = jnp.zeros((E,), i32).at[ids].add(1)
    padded = ((counts + TILE - 1) // TILE) * TILE
    seg_end = jnp.cumsum(padded)
    seg_start = seg_end - padded
    unp_start = jnp.cumsum(counts) - counts
    pos = seg_start[sorted_ids] + (
        jnp.arange(2 * N, dtype=i32) - unp_start[sorted_ids])
    pad_tok = jnp.arange(P, dtype=i32) % N
    perm_token = pad_tok.at[pos].set((order // 2).astype(i32))
    w_sorted = jnp.zeros((P,), jnp.float32).at[pos].set(wfl[order])
    inv = jnp.zeros((2 * N,), i32).at[order].set(pos)
    tile_expert = jnp.minimum(
        jnp.searchsorted(seg_end, jnp.arange(NTT, dtype=i32) * TILE,
                         side="right"),
        E - 1).astype(i32)

    x_s = _row_gather(xf, perm_token, P)

    grid_spec_up = pltpu.PrefetchScalarGridSpec(
        num_scalar_prefetch=1,
        grid=(NF, NTT),
        in_specs=[
            pl.BlockSpec((TILE, D), lambda f, t, te: (t, 0)),
            pl.BlockSpec((1, F, D), lambda f, t, te: (te[t], f, 0)),
            pl.BlockSpec((1, F, D), lambda f, t, te: (te[t], NF + f, 0)),
        ],
        out_specs=pl.BlockSpec((TILE, F), lambda f, t, te: (t, f)),
    )
    h_s = pl.pallas_call(
        _up_body,
        grid_spec=grid_spec_up,
        out_shape=jax.ShapeDtypeStruct((P, half), jnp.float32),
        compiler_params=pltpu.CompilerParams(
            dimension_semantics=("arbitrary", "arbitrary")),
    )(tile_expert, x_s, w_up, w_up)

    ws_r = w_sorted.reshape(NTT, TILE, 1)
    grid_spec_dn = pltpu.PrefetchScalarGridSpec(
        num_scalar_prefetch=1,
        grid=(NTT,),
        in_specs=[
            pl.BlockSpec((TILE, half), lambda t, te: (t, 0)),
            pl.BlockSpec((1, D, half), lambda t, te: (te[t], 0, 0)),
            pl.BlockSpec((1, TILE, 1), lambda t, te: (t, 0, 0)),
        ],
        out_specs=pl.BlockSpec((TILE, D), lambda t, te: (t, 0)),
    )
    out_s = pl.pallas_call(
        _down_body,
        grid_spec=grid_spec_dn,
        out_shape=jax.ShapeDtypeStruct((P, D), jnp.float32),
        compiler_params=pltpu.CompilerParams(
            dimension_semantics=("arbitrary",)),
    )(tile_expert, h_s, w_down, ws_r)

    rpair = _row_gather(out_s, inv, 2 * N).reshape(N, 2 * D)

    NFS = 8 if half % 8 == 0 else 1
    FS = half // NFS
    out = pl.pallas_call(
        _final_body,
        grid=(NT, NFS),
        in_specs=[
            pl.BlockSpec((TB, D), lambda t, f: (t, 0)),
            pl.BlockSpec((FS, D), lambda t, f: (f, 0)),
            pl.BlockSpec((FS, D), lambda t, f: (f, 0)),
            pl.BlockSpec((D, FS), lambda t, f: (0, f)),
            pl.BlockSpec((TB, 2 * D), lambda t, f: (t, 0)),
        ],
        out_specs=pl.BlockSpec((TB, D), lambda t, f: (t, 0)),
        out_shape=jax.ShapeDtypeStruct((N, D), jnp.float32),
        compiler_params=pltpu.CompilerParams(
            dimension_semantics=("parallel", "arbitrary")),
    )(xf, sw_gate, sw_up, sw_down, rpair)

    return out.reshape(B, T, D)


# speedup vs baseline: 1.0021x; 1.0021x over previous
import functools

import jax
import jax.numpy as jnp
from jax import lax
from jax.experimental import pallas as pl
from jax.experimental.pallas import tpu as pltpu
from jax.experimental.pallas import tpu_sc as plsc


def _row_gather(src, idx, P):
    R, D = src.shape
    info = plsc.get_sparse_core_info()
    NW = info.num_cores * info.num_subcores
    rpw = P // NW
    C = 16
    while rpw % C:
        C //= 2
    nch = rpw // C
    mesh = plsc.VectorSubcoreMesh(core_axis_name="c", subcore_axis_name="s")

    @functools.partial(
        pl.kernel, mesh=mesh,
        out_type=jax.ShapeDtypeStruct((P, D), src.dtype),
        scratch_types=[
            pltpu.VMEM((rpw,), jnp.int32),
            pltpu.VMEM((C, D), src.dtype),
            pltpu.VMEM((C, D), src.dtype),
            pltpu.SemaphoreType.DMA,
            pltpu.SemaphoreType.DMA,
        ])
    def gk(src_hbm, idx_hbm, out_hbm, idx_v, buf0, buf1, sem0, sem1):
        wid = lax.axis_index("s") * info.num_cores + lax.axis_index("c")
        base = wid * rpw
        pltpu.sync_copy(idx_hbm.at[pl.ds(base, rpw)], idx_v)
        bufs = (buf0, buf1)
        sems = (sem0, sem1)
        pltpu.async_copy(src_hbm.at[idx_v.at[pl.ds(0, C)]], buf0, sem0)
        for i in range(nch):
            b, s = bufs[i % 2], sems[i % 2]
            pltpu.make_async_copy(
                src_hbm.at[idx_v.at[pl.ds(i * C, C)]], b, s).wait()
            if i + 1 < nch:
                pltpu.async_copy(
                    src_hbm.at[idx_v.at[pl.ds((i + 1) * C, C)]],
                    bufs[(i + 1) % 2], sems[(i + 1) % 2])
            pltpu.sync_copy(b, out_hbm.at[pl.ds(base + i * C, C)])

    return gk(src, idx)


def _row_scatter(src, idx, PO):
    P, D = src.shape
    info = plsc.get_sparse_core_info()
    NW = info.num_cores * info.num_subcores
    rpw = P // NW
    C = 16
    while rpw % C:
        C //= 2
    nch = rpw // C
    idx_r = idx.reshape(NW, nch, C)
    mesh = plsc.VectorSubcoreMesh(core_axis_name="c", subcore_axis_name="s")

    @functools.partial(
        pl.kernel, mesh=mesh,
        out_type=jax.ShapeDtypeStruct((PO, D), src.dtype),
        scratch_types=[
            pltpu.VMEM((nch, C), jnp.int32),
            pltpu.VMEM((C, D), src.dtype),
            pltpu.VMEM((C, D), src.dtype),
            pltpu.SemaphoreType.DMA,
            pltpu.SemaphoreType.DMA,
            pltpu.SemaphoreType.DMA,
        ])
    def sk(src_hbm, idx_hbm, out_hbm, idx_v, buf0, buf1, lsem0, lsem1, ssem):
        wid = lax.axis_index("s") * info.num_cores + lax.axis_index("c")
        base = wid * rpw
        pltpu.sync_copy(idx_hbm.at[wid], idx_v)
        bufs = (buf0, buf1)
        sems = (lsem0, lsem1)
        pltpu.async_copy(src_hbm.at[pl.ds(base, C)], buf0, lsem0)
        for i in range(nch):
            b, s = bufs[i % 2], sems[i % 2]
            pltpu.make_async_copy(
                src_hbm.at[pl.ds(base + i * C, C)], b, s).wait()
            if i + 1 < nch:
                pltpu.async_copy(
                    src_hbm.at[pl.ds(base + (i + 1) * C, C)],
                    bufs[(i + 1) % 2], sems[(i + 1) % 2])
            pltpu.async_copy(b, out_hbm.at[idx_v.at[i]], ssem)
            pltpu.make_async_copy(b, out_hbm.at[idx_v.at[i]], ssem).wait()

    return sk(src, idx_r)


def _dot_t(a, b):
    return jax.lax.dot_general(a, b, (((1,), (1,)), ((), ())),
                               preferred_element_type=jnp.float32)


def _dot_t_bf(a, b):
    return jax.lax.dot_general(a.astype(jnp.bfloat16), b.astype(jnp.bfloat16),
                               (((1,), (1,)), ((), ())),
                               preferred_element_type=jnp.float32)


def _router_body(x_ref, gw_ref, idx_ref, w_ref):
    x = x_ref[...]
    gw = gw_ref[...]
    logits = _dot_t(x, gw)
    e_num = logits.shape[1]
    iota = jax.lax.broadcasted_iota(jnp.int32, logits.shape, 1)
    m1 = jnp.max(logits, axis=1, keepdims=True)
    cand1 = jnp.where(logits == m1, iota, e_num)
    i1 = jnp.min(cand1, axis=1, keepdims=True)
    mask1 = iota == i1
    l2 = jnp.where(mask1, -jnp.inf, logits)
    m2 = jnp.max(l2, axis=1, keepdims=True)
    cand2 = jnp.where(l2 == m2, iota, e_num)
    i2 = jnp.min(cand2, axis=1, keepdims=True)
    b = jnp.exp(m2 - m1)
    w1 = 1.0 / (1.0 + b)
    w2 = 1.0 - w1
    idx_ref[...] = jnp.concatenate([i1, i2], axis=1)
    w_ref[...] = jnp.concatenate([w1, w2], axis=1)


def _up_body(te_ref, x_ref, wg_ref, wu_ref, h_ref):
    x = x_ref[...]
    g = _dot_t_bf(x, wg_ref[0])
    u = _dot_t_bf(x, wu_ref[0])
    h_ref[...] = (g * jax.nn.sigmoid(g)) * u


def _down_body(te_ref, h_ref, wd_ref, ws_ref, out_ref):
    h = h_ref[...]
    d = _dot_t_bf(h, wd_ref[0])
    out_ref[...] = ws_ref[0] * d


def _shared_body(x_ref, sg_ref, su_ref, sd_ref, out_ref):
    f = pl.program_id(1)
    x = x_ref[...]
    g = _dot_t_bf(x, sg_ref[...])
    u = _dot_t_bf(x, su_ref[...])
    h = (g * jax.nn.sigmoid(g)) * u
    d = _dot_t_bf(h, sd_ref[...])

    @pl.when(f == 0)
    def _():
        out_ref[...] = d

    @pl.when(f != 0)
    def _():
        out_ref[...] += d


def _add_body(rp_ref, s_ref, out_ref):
    rp = rp_ref[...]
    dd = s_ref.shape[1]
    out_ref[...] = rp[:, :dd] + rp[:, dd:] + s_ref[...]


def kernel(x, gate_w, w_up, w_down, sw_gate, sw_up, sw_down):
    B, T, D = x.shape
    N = B * T
    E = gate_w.shape[0]
    half = w_down.shape[2]
    xf = x.reshape(N, D)

    TB = 512 if N % 512 == 0 else N
    NT = N // TB
    TILE = 256 if N >= 2048 else 64
    NTT = (2 * N) // TILE + E
    P = NTT * TILE
    NF = 2 if half % 256 == 0 else 1
    F = half // NF

    topk_idx, topk_w = pl.pallas_call(
        _router_body,
        grid=(NT,),
        in_specs=[
            pl.BlockSpec((TB, D), lambda t: (t, 0)),
            pl.BlockSpec((E, D), lambda t: (0, 0)),
        ],
        out_specs=[
            pl.BlockSpec((TB, 2), lambda t: (t, 0)),
            pl.BlockSpec((TB, 2), lambda t: (t, 0)),
        ],
        out_shape=[
            jax.ShapeDtypeStruct((N, 2), jnp.int32),
            jax.ShapeDtypeStruct((N, 2), jnp.float32),
        ],
    )(xf, gate_w)

    NFS = 8 if half % 8 == 0 else 1
    FS = half // NFS
    shared = pl.pallas_call(
        _shared_body,
        grid=(NT, NFS),
        in_specs=[
            pl.BlockSpec((TB, D), lambda t, f: (t, 0)),
            pl.BlockSpec((FS, D), lambda t, f: (f, 0)),
            pl.BlockSpec((FS, D), lambda t, f: (f, 0)),
            pl.BlockSpec((D, FS), lambda t, f: (0, f)),
        ],
        out_specs=pl.BlockSpec((TB, D), lambda t, f: (t, 0)),
        out_shape=jax.ShapeDtypeStruct((N, D), jnp.float32),
        compiler_params=pltpu.CompilerParams(
            dimension_semantics=("parallel", "arbitrary")),
    )(xf, sw_gate, sw_up, sw_down)

    i32 = jnp.int32
    ids = topk_idx.reshape(-1)
    wfl = topk_w.reshape(-1)
    order = jnp.argsort(ids, stable=True)
    sorted_ids = ids[order]
    counts = jnp.zeros((E,), i32).at[ids].add(1)
    padded = ((counts + TILE - 1) // TILE) * TILE
    seg_end = jnp.cumsum(padded)
    seg_start = seg_end - padded
    unp_start = jnp.cumsum(counts) - counts
    pos = seg_start[sorted_ids] + (
        jnp.arange(2 * N, dtype=i32) - unp_start[sorted_ids])
    pad_tok = jnp.arange(P, dtype=i32) % N
    perm_token = pad_tok.at[pos].set((order // 2).astype(i32))
    w_sorted = jnp.zeros((P,), jnp.float32).at[pos].set(wfl[order])
    inv = jnp.zeros((2 * N,), i32).at[order].set(pos)
    tile_expert = jnp.minimum(
        jnp.searchsorted(seg_end, jnp.arange(NTT, dtype=i32) * TILE,
                         side="right"),
        E - 1).astype(i32)

    x_s = _row_gather(xf, perm_token, P)

    grid_spec_up = pltpu.PrefetchScalarGridSpec(
        num_scalar_prefetch=1,
        grid=(NF, NTT),
        in_specs=[
            pl.BlockSpec((TILE, D), lambda f, t, te: (t, 0)),
            pl.BlockSpec((1, F, D), lambda f, t, te: (te[t], f, 0)),
            pl.BlockSpec((1, F, D), lambda f, t, te: (te[t], NF + f, 0)),
        ],
        out_specs=pl.BlockSpec((TILE, F), lambda f, t, te: (t, f)),
    )
    h_s = pl.pallas_call(
        _up_body,
        grid_spec=grid_spec_up,
        out_shape=jax.ShapeDtypeStruct((P, half), jnp.float32),
        compiler_params=pltpu.CompilerParams(
            dimension_semantics=("arbitrary", "arbitrary")),
    )(tile_expert, x_s, w_up, w_up)

    ws_r = w_sorted.reshape(NTT, TILE, 1)
    grid_spec_dn = pltpu.PrefetchScalarGridSpec(
        num_scalar_prefetch=1,
        grid=(NTT,),
        in_specs=[
            pl.BlockSpec((TILE, half), lambda t, te: (t, 0)),
            pl.BlockSpec((1, D, half), lambda t, te: (te[t], 0, 0)),
            pl.BlockSpec((1, TILE, 1), lambda t, te: (t, 0, 0)),
        ],
        out_specs=pl.BlockSpec((TILE, D), lambda t, te: (t, 0)),
    )
    out_s = pl.pallas_call(
        _down_body,
        grid_spec=grid_spec_dn,
        out_shape=jax.ShapeDtypeStruct((P, D), jnp.float32),
        compiler_params=pltpu.CompilerParams(
            dimension_semantics=("arbitrary",)),
    )(tile_expert, h_s, w_down, ws_r)

    rpair = _row_gather(out_s, inv, 2 * N).reshape(N, 2 * D)

    out = pl.pallas_call(
        _add_body,
        grid=(NT,),
        in_specs=[
            pl.BlockSpec((TB, 2 * D), lambda t: (t, 0)),
            pl.BlockSpec((TB, D), lambda t: (t, 0)),
        ],
        out_specs=pl.BlockSpec((TB, D), lambda t: (t, 0)),
        out_shape=jax.ShapeDtypeStruct((N, D), jnp.float32),
        compiler_params=pltpu.CompilerParams(
            dimension_semantics=("parallel",)),
    )(rpair, shared)

    return out.reshape(B, T, D)


# speedup vs baseline: 1.0331x; 1.0309x over previous
import functools

import jax
import jax.numpy as jnp
from jax import lax
from jax.experimental import pallas as pl
from jax.experimental.pallas import tpu as pltpu
from jax.experimental.pallas import tpu_sc as plsc


def _row_gather(src, idx, P):
    R, D = src.shape
    info = plsc.get_sparse_core_info()
    NW = info.num_cores * info.num_subcores
    rpw = P // NW
    C = 16
    while rpw % C:
        C //= 2
    nch = rpw // C
    mesh = plsc.VectorSubcoreMesh(core_axis_name="c", subcore_axis_name="s")

    @functools.partial(
        pl.kernel, mesh=mesh,
        out_type=jax.ShapeDtypeStruct((P, D), src.dtype),
        scratch_types=[
            pltpu.VMEM((rpw,), jnp.int32),
            pltpu.VMEM((C, D), src.dtype),
            pltpu.VMEM((C, D), src.dtype),
            pltpu.SemaphoreType.DMA,
            pltpu.SemaphoreType.DMA,
        ])
    def gk(src_hbm, idx_hbm, out_hbm, idx_v, buf0, buf1, sem0, sem1):
        wid = lax.axis_index("s") * info.num_cores + lax.axis_index("c")
        base = wid * rpw
        pltpu.sync_copy(idx_hbm.at[pl.ds(base, rpw)], idx_v)
        bufs = (buf0, buf1)
        sems = (sem0, sem1)
        pltpu.async_copy(src_hbm.at[idx_v.at[pl.ds(0, C)]], buf0, sem0)
        for i in range(nch):
            b, s = bufs[i % 2], sems[i % 2]
            pltpu.make_async_copy(
                src_hbm.at[idx_v.at[pl.ds(i * C, C)]], b, s).wait()
            if i + 1 < nch:
                pltpu.async_copy(
                    src_hbm.at[idx_v.at[pl.ds((i + 1) * C, C)]],
                    bufs[(i + 1) % 2], sems[(i + 1) % 2])
            pltpu.sync_copy(b, out_hbm.at[pl.ds(base + i * C, C)])

    return gk(src, idx)


def _row_scatter(src, idx, PO):
    P, D = src.shape
    info = plsc.get_sparse_core_info()
    NW = info.num_cores * info.num_subcores
    rpw = P // NW
    C = 16
    while rpw % C:
        C //= 2
    nch = rpw // C
    idx_r = idx.reshape(NW, nch, C)
    mesh = plsc.VectorSubcoreMesh(core_axis_name="c", subcore_axis_name="s")

    @functools.partial(
        pl.kernel, mesh=mesh,
        out_type=jax.ShapeDtypeStruct((PO, D), src.dtype),
        scratch_types=[
            pltpu.VMEM((nch, C), jnp.int32),
            pltpu.VMEM((C, D), src.dtype),
            pltpu.VMEM((C, D), src.dtype),
            pltpu.SemaphoreType.DMA,
            pltpu.SemaphoreType.DMA,
            pltpu.SemaphoreType.DMA,
        ])
    def sk(src_hbm, idx_hbm, out_hbm, idx_v, buf0, buf1, lsem0, lsem1, ssem):
        wid = lax.axis_index("s") * info.num_cores + lax.axis_index("c")
        base = wid * rpw
        pltpu.sync_copy(idx_hbm.at[wid], idx_v)
        bufs = (buf0, buf1)
        sems = (lsem0, lsem1)
        pltpu.async_copy(src_hbm.at[pl.ds(base, C)], buf0, lsem0)
        for i in range(nch):
            b, s = bufs[i % 2], sems[i % 2]
            pltpu.make_async_copy(
                src_hbm.at[pl.ds(base + i * C, C)], b, s).wait()
            if i + 1 < nch:
                pltpu.async_copy(
                    src_hbm.at[pl.ds(base + (i + 1) * C, C)],
                    bufs[(i + 1) % 2], sems[(i + 1) % 2])
            pltpu.async_copy(b, out_hbm.at[idx_v.at[i]], ssem)
            pltpu.make_async_copy(b, out_hbm.at[idx_v.at[i]], ssem).wait()

    return sk(src, idx_r)


def _dot_t(a, b):
    return jax.lax.dot_general(a, b, (((1,), (1,)), ((), ())),
                               preferred_element_type=jnp.float32)


def _dot_t_bf(a, b):
    return jax.lax.dot_general(a.astype(jnp.bfloat16), b.astype(jnp.bfloat16),
                               (((1,), (1,)), ((), ())),
                               preferred_element_type=jnp.float32)


def _router_body(x_ref, gw_ref, idx_ref, w_ref):
    x = x_ref[...]
    gw = gw_ref[...]
    logits = _dot_t(x, gw)
    e_num = logits.shape[1]
    iota = jax.lax.broadcasted_iota(jnp.int32, logits.shape, 1)
    m1 = jnp.max(logits, axis=1, keepdims=True)
    cand1 = jnp.where(logits == m1, iota, e_num)
    i1 = jnp.min(cand1, axis=1, keepdims=True)
    mask1 = iota == i1
    l2 = jnp.where(mask1, -jnp.inf, logits)
    m2 = jnp.max(l2, axis=1, keepdims=True)
    cand2 = jnp.where(l2 == m2, iota, e_num)
    i2 = jnp.min(cand2, axis=1, keepdims=True)
    b = jnp.exp(m2 - m1)
    w1 = 1.0 / (1.0 + b)
    w2 = 1.0 - w1
    idx_ref[...] = jnp.concatenate([i1, i2], axis=1)
    w_ref[...] = jnp.concatenate([w1, w2], axis=1)


def _up_body(te_ref, act_ref, x_ref, wg_ref, wu_ref, h_ref):
    @pl.when(pl.program_id(1) < act_ref[0])
    def _():
        x = x_ref[...]
        g = _dot_t_bf(x, wg_ref[0])
        u = _dot_t_bf(x, wu_ref[0])
        h_ref[...] = (g * jax.nn.sigmoid(g)) * u


def _down_body(te_ref, act_ref, h_ref, wd_ref, ws_ref, out_ref):
    @pl.when(pl.program_id(0) < act_ref[0])
    def _():
        h = h_ref[...]
        d = _dot_t_bf(h, wd_ref[0])
        out_ref[...] = ws_ref[0] * d


def _shared_body(x_ref, sg_ref, su_ref, sd_ref, out_ref):
    f = pl.program_id(1)
    x = x_ref[...]
    g = _dot_t_bf(x, sg_ref[...])
    u = _dot_t_bf(x, su_ref[...])
    h = (g * jax.nn.sigmoid(g)) * u
    d = _dot_t_bf(h, sd_ref[...])

    @pl.when(f == 0)
    def _():
        out_ref[...] = d

    @pl.when(f != 0)
    def _():
        out_ref[...] += d


def _add_body(rp_ref, s_ref, out_ref):
    rp = rp_ref[...]
    dd = s_ref.shape[1]
    out_ref[...] = rp[:, :dd] + rp[:, dd:] + s_ref[...]


def kernel(x, gate_w, w_up, w_down, sw_gate, sw_up, sw_down):
    B, T, D = x.shape
    N = B * T
    E = gate_w.shape[0]
    half = w_down.shape[2]
    xf = x.reshape(N, D)

    TB = 512 if N % 512 == 0 else N
    NT = N // TB
    TILE = 256 if N >= 2048 else 64
    NTT = (2 * N) // TILE + E
    P = NTT * TILE
    NF = 2 if half % 256 == 0 else 1
    F = half // NF

    topk_idx, topk_w = pl.pallas_call(
        _router_body,
        grid=(NT,),
        in_specs=[
            pl.BlockSpec((TB, D), lambda t: (t, 0)),
            pl.BlockSpec((E, D), lambda t: (0, 0)),
        ],
        out_specs=[
            pl.BlockSpec((TB, 2), lambda t: (t, 0)),
            pl.BlockSpec((TB, 2), lambda t: (t, 0)),
        ],
        out_shape=[
            jax.ShapeDtypeStruct((N, 2), jnp.int32),
            jax.ShapeDtypeStruct((N, 2), jnp.float32),
        ],
    )(xf, gate_w)

    NFS = 8 if half % 8 == 0 else 1
    FS = half // NFS
    shared = pl.pallas_call(
        _shared_body,
        grid=(NT, NFS),
        in_specs=[
            pl.BlockSpec((TB, D), lambda t, f: (t, 0)),
            pl.BlockSpec((FS, D), lambda t, f: (f, 0)),
            pl.BlockSpec((FS, D), lambda t, f: (f, 0)),
            pl.BlockSpec((D, FS), lambda t, f: (0, f)),
        ],
        out_specs=pl.BlockSpec((TB, D), lambda t, f: (t, 0)),
        out_shape=jax.ShapeDtypeStruct((N, D), jnp.float32),
        compiler_params=pltpu.CompilerParams(
            dimension_semantics=("parallel", "arbitrary")),
    )(xf, sw_gate, sw_up, sw_down)

    i32 = jnp.int32
    ids = topk_idx.reshape(-1)
    wfl = topk_w.reshape(-1)
    order = jnp.argsort(ids, stable=True)
    sorted_ids = ids[order]
    counts = jnp.zeros((E,), i32).at[ids].add(1)
    padded = ((counts + TILE - 1) // TILE) * TILE
    seg_end = jnp.cumsum(padded)
    seg_start = seg_end - padded
    unp_start = jnp.cumsum(counts) - counts
    pos = seg_start[sorted_ids] + (
        jnp.arange(2 * N, dtype=i32) - unp_start[sorted_ids])
    pad_tok = jnp.arange(P, dtype=i32) % N
    perm_token = pad_tok.at[pos].set((order // 2).astype(i32))
    w_sorted = jnp.zeros((P,), jnp.float32).at[pos].set(wfl[order])
    inv = jnp.zeros((2 * N,), i32).at[order].set(pos)
    act = (seg_end[E - 1] // TILE).astype(i32).reshape(1)
    tile_expert = jnp.minimum(
        jnp.searchsorted(seg_end, jnp.arange(NTT, dtype=i32) * TILE,
                         side="right"),
        E - 1).astype(i32)

    x_s = _row_gather(xf, perm_token, P)

    grid_spec_up = pltpu.PrefetchScalarGridSpec(
        num_scalar_prefetch=2,
        grid=(NF, NTT),
        in_specs=[
            pl.BlockSpec(
                (TILE, D),
                lambda f, t, te, act: (jnp.minimum(t, act[0] - 1), 0)),
            pl.BlockSpec(
                (1, F, D),
                lambda f, t, te, act: (te[jnp.minimum(t, act[0] - 1)], f, 0)),
            pl.BlockSpec(
                (1, F, D),
                lambda f, t, te, act: (te[jnp.minimum(t, act[0] - 1)],
                                       NF + f, 0)),
        ],
        out_specs=pl.BlockSpec(
            (TILE, F), lambda f, t, te, act: (jnp.minimum(t, act[0] - 1), f)),
    )
    h_s = pl.pallas_call(
        _up_body,
        grid_spec=grid_spec_up,
        out_shape=jax.ShapeDtypeStruct((P, half), jnp.float32),
        compiler_params=pltpu.CompilerParams(
            dimension_semantics=("arbitrary", "arbitrary")),
    )(tile_expert, act, x_s, w_up, w_up)

    ws_r = w_sorted.reshape(NTT, TILE, 1)
    grid_spec_dn = pltpu.PrefetchScalarGridSpec(
        num_scalar_prefetch=2,
        grid=(NTT,),
        in_specs=[
            pl.BlockSpec(
                (TILE, half),
                lambda t, te, act: (jnp.minimum(t, act[0] - 1), 0)),
            pl.BlockSpec(
                (1, D, half),
                lambda t, te, act: (te[jnp.minimum(t, act[0] - 1)], 0, 0)),
            pl.BlockSpec(
                (1, TILE, 1),
                lambda t, te, act: (jnp.minimum(t, act[0] - 1), 0, 0)),
        ],
        out_specs=pl.BlockSpec(
            (TILE, D), lambda t, te, act: (jnp.minimum(t, act[0] - 1), 0)),
    )
    out_s = pl.pallas_call(
        _down_body,
        grid_spec=grid_spec_dn,
        out_shape=jax.ShapeDtypeStruct((P, D), jnp.float32),
        compiler_params=pltpu.CompilerParams(
            dimension_semantics=("arbitrary",)),
    )(tile_expert, act, h_s, w_down, ws_r)

    rpair = _row_gather(out_s, inv, 2 * N).reshape(N, 2 * D)

    out = pl.pallas_call(
        _add_body,
        grid=(NT,),
        in_specs=[
            pl.BlockSpec((TB, 2 * D), lambda t: (t, 0)),
            pl.BlockSpec((TB, D), lambda t: (t, 0)),
        ],
        out_specs=pl.BlockSpec((TB, D), lambda t: (t, 0)),
        out_shape=jax.ShapeDtypeStruct((N, D), jnp.float32),
        compiler_params=pltpu.CompilerParams(
            dimension_semantics=("parallel",)),
    )(rpair, shared)

    return out.reshape(B, T, D)


# speedup vs baseline: 1.0691x; 1.0349x over previous
import functools

import jax
import jax.numpy as jnp
from jax import lax
from jax.experimental import pallas as pl
from jax.experimental.pallas import tpu as pltpu
from jax.experimental.pallas import tpu_sc as plsc


def _row_gather(src, idx, P):
    R, D = src.shape
    info = plsc.get_sparse_core_info()
    NW = info.num_cores * info.num_subcores
    rpw = P // NW
    C = 16
    while rpw % C:
        C //= 2
    nch = rpw // C
    mesh = plsc.VectorSubcoreMesh(core_axis_name="c", subcore_axis_name="s")

    @functools.partial(
        pl.kernel, mesh=mesh,
        out_type=jax.ShapeDtypeStruct((P, D), src.dtype),
        scratch_types=[
            pltpu.VMEM((rpw,), jnp.int32),
            pltpu.VMEM((C, D), src.dtype),
            pltpu.VMEM((C, D), src.dtype),
            pltpu.SemaphoreType.DMA,
            pltpu.SemaphoreType.DMA,
        ])
    def gk(src_hbm, idx_hbm, out_hbm, idx_v, buf0, buf1, sem0, sem1):
        wid = lax.axis_index("s") * info.num_cores + lax.axis_index("c")
        base = wid * rpw
        pltpu.sync_copy(idx_hbm.at[pl.ds(base, rpw)], idx_v)
        bufs = (buf0, buf1)
        sems = (sem0, sem1)
        pltpu.async_copy(src_hbm.at[idx_v.at[pl.ds(0, C)]], buf0, sem0)
        for i in range(nch):
            b, s = bufs[i % 2], sems[i % 2]
            pltpu.make_async_copy(
                src_hbm.at[idx_v.at[pl.ds(i * C, C)]], b, s).wait()
            if i + 1 < nch:
                pltpu.async_copy(
                    src_hbm.at[idx_v.at[pl.ds((i + 1) * C, C)]],
                    bufs[(i + 1) % 2], sems[(i + 1) % 2])
            pltpu.sync_copy(b, out_hbm.at[pl.ds(base + i * C, C)])

    return gk(src, idx)


def _row_scatter(src, idx, PO):
    P, D = src.shape
    info = plsc.get_sparse_core_info()
    NW = info.num_cores * info.num_subcores
    rpw = P // NW
    C = 16
    while rpw % C:
        C //= 2
    nch = rpw // C
    idx_r = idx.reshape(NW, nch, C)
    mesh = plsc.VectorSubcoreMesh(core_axis_name="c", subcore_axis_name="s")

    @functools.partial(
        pl.kernel, mesh=mesh,
        out_type=jax.ShapeDtypeStruct((PO, D), src.dtype),
        scratch_types=[
            pltpu.VMEM((nch, C), jnp.int32),
            pltpu.VMEM((C, D), src.dtype),
            pltpu.VMEM((C, D), src.dtype),
            pltpu.SemaphoreType.DMA,
            pltpu.SemaphoreType.DMA,
            pltpu.SemaphoreType.DMA,
        ])
    def sk(src_hbm, idx_hbm, out_hbm, idx_v, buf0, buf1, lsem0, lsem1, ssem):
        wid = lax.axis_index("s") * info.num_cores + lax.axis_index("c")
        base = wid * rpw
        pltpu.sync_copy(idx_hbm.at[wid], idx_v)
        bufs = (buf0, buf1)
        sems = (lsem0, lsem1)
        pltpu.async_copy(src_hbm.at[pl.ds(base, C)], buf0, lsem0)
        for i in range(nch):
            b, s = bufs[i % 2], sems[i % 2]
            pltpu.make_async_copy(
                src_hbm.at[pl.ds(base + i * C, C)], b, s).wait()
            if i + 1 < nch:
                pltpu.async_copy(
                    src_hbm.at[pl.ds(base + (i + 1) * C, C)],
                    bufs[(i + 1) % 2], sems[(i + 1) % 2])
            pltpu.async_copy(b, out_hbm.at[idx_v.at[i]], ssem)
            pltpu.make_async_copy(b, out_hbm.at[idx_v.at[i]], ssem).wait()

    return sk(src, idx_r)


def _dot_t(a, b):
    return jax.lax.dot_general(a, b, (((1,), (1,)), ((), ())),
                               preferred_element_type=jnp.float32)


def _dot_t_bf(a, b):
    return jax.lax.dot_general(a.astype(jnp.bfloat16), b.astype(jnp.bfloat16),
                               (((1,), (1,)), ((), ())),
                               preferred_element_type=jnp.float32)


def _router_body(x_ref, gw_ref, idx_ref, w_ref):
    x = x_ref[...]
    gw = gw_ref[...]
    logits = _dot_t(x, gw)
    e_num = logits.shape[1]
    iota = jax.lax.broadcasted_iota(jnp.int32, logits.shape, 1)
    m1 = jnp.max(logits, axis=1, keepdims=True)
    cand1 = jnp.where(logits == m1, iota, e_num)
    i1 = jnp.min(cand1, axis=1, keepdims=True)
    mask1 = iota == i1
    l2 = jnp.where(mask1, -jnp.inf, logits)
    m2 = jnp.max(l2, axis=1, keepdims=True)
    cand2 = jnp.where(l2 == m2, iota, e_num)
    i2 = jnp.min(cand2, axis=1, keepdims=True)
    b = jnp.exp(m2 - m1)
    w1 = 1.0 / (1.0 + b)
    w2 = 1.0 - w1
    idx_ref[...] = jnp.concatenate([i1, i2], axis=1)
    w_ref[...] = jnp.concatenate([w1, w2], axis=1)


def _up_body(te_ref, act_ref, x_ref, wg_ref, wu_ref, h_ref):
    @pl.when(pl.program_id(1) < act_ref[0])
    def _():
        x = x_ref[...]
        g = _dot_t_bf(x, wg_ref[0])
        u = _dot_t_bf(x, wu_ref[0])
        h_ref[...] = (g * jax.nn.sigmoid(g)) * u


def _down_body(te_ref, act_ref, h_ref, wd_ref, ws_ref, out_ref):
    @pl.when(pl.program_id(0) < act_ref[0])
    def _():
        h = h_ref[...]
        d = _dot_t_bf(h, wd_ref[0])
        out_ref[...] = ws_ref[0] * d


def _shared_body(x_ref, sg_ref, su_ref, sd_ref, out_ref):
    f = pl.program_id(1)
    x = x_ref[...]
    g = _dot_t_bf(x, sg_ref[...])
    u = _dot_t_bf(x, su_ref[...])
    h = (g * jax.nn.sigmoid(g)) * u
    d = _dot_t_bf(h, sd_ref[...])

    @pl.when(f == 0)
    def _():
        out_ref[...] = d

    @pl.when(f != 0)
    def _():
        out_ref[...] += d


def _add_body(rp_ref, s_ref, out_ref):
    rp = rp_ref[...]
    dd = s_ref.shape[1]
    out_ref[...] = rp[:, :dd] + rp[:, dd:] + s_ref[...]


def kernel(x, gate_w, w_up, w_down, sw_gate, sw_up, sw_down):
    B, T, D = x.shape
    N = B * T
    E = gate_w.shape[0]
    half = w_down.shape[2]
    xf = x.reshape(N, D)

    TB = 512 if N % 512 == 0 else N
    NT = N // TB
    TILE = 512 if N >= 2048 else 64
    NTT = (2 * N) // TILE + E
    P = NTT * TILE
    NF = 2 if half % 256 == 0 else 1
    F = half // NF

    topk_idx, topk_w = pl.pallas_call(
        _router_body,
        grid=(NT,),
        in_specs=[
            pl.BlockSpec((TB, D), lambda t: (t, 0)),
            pl.BlockSpec((E, D), lambda t: (0, 0)),
        ],
        out_specs=[
            pl.BlockSpec((TB, 2), lambda t: (t, 0)),
            pl.BlockSpec((TB, 2), lambda t: (t, 0)),
        ],
        out_shape=[
            jax.ShapeDtypeStruct((N, 2), jnp.int32),
            jax.ShapeDtypeStruct((N, 2), jnp.float32),
        ],
    )(xf, gate_w)

    NFS = 8 if half % 8 == 0 else 1
    FS = half // NFS
    shared = pl.pallas_call(
        _shared_body,
        grid=(NT, NFS),
        in_specs=[
            pl.BlockSpec((TB, D), lambda t, f: (t, 0)),
            pl.BlockSpec((FS, D), lambda t, f: (f, 0)),
            pl.BlockSpec((FS, D), lambda t, f: (f, 0)),
            pl.BlockSpec((D, FS), lambda t, f: (0, f)),
        ],
        out_specs=pl.BlockSpec((TB, D), lambda t, f: (t, 0)),
        out_shape=jax.ShapeDtypeStruct((N, D), jnp.float32),
        compiler_params=pltpu.CompilerParams(
            dimension_semantics=("parallel", "arbitrary")),
    )(xf, sw_gate, sw_up, sw_down)

    i32 = jnp.int32
    ids = topk_idx.reshape(-1)
    wfl = topk_w.reshape(-1)
    order = jnp.argsort(ids, stable=True)
    sorted_ids = ids[order]
    counts = jnp.zeros((E,), i32).at[ids].add(1)
    padded = ((counts + TILE - 1) // TILE) * TILE
    seg_end = jnp.cumsum(padded)
    seg_start = seg_end - padded
    unp_start = jnp.cumsum(counts) - counts
    pos = seg_start[sorted_ids] + (
        jnp.arange(2 * N, dtype=i32) - unp_start[sorted_ids])
    pad_tok = jnp.arange(P, dtype=i32) % N
    perm_token = pad_tok.at[pos].set((order // 2).astype(i32))
    w_sorted = jnp.zeros((P,), jnp.float32).at[pos].set(wfl[order])
    inv = jnp.zeros((2 * N,), i32).at[order].set(pos)
    act = (seg_end[E - 1] // TILE).astype(i32).reshape(1)
    tile_expert = jnp.minimum(
        jnp.searchsorted(seg_end, jnp.arange(NTT, dtype=i32) * TILE,
                         side="right"),
        E - 1).astype(i32)

    x_s = _row_gather(xf, perm_token, P)

    grid_spec_up = pltpu.PrefetchScalarGridSpec(
        num_scalar_prefetch=2,
        grid=(NF, NTT),
        in_specs=[
            pl.BlockSpec(
                (TILE, D),
                lambda f, t, te, act: (jnp.minimum(t, act[0] - 1), 0)),
            pl.BlockSpec(
                (1, F, D),
                lambda f, t, te, act: (te[jnp.minimum(t, act[0] - 1)], f, 0)),
            pl.BlockSpec(
                (1, F, D),
                lambda f, t, te, act: (te[jnp.minimum(t, act[0] - 1)],
                                       NF + f, 0)),
        ],
        out_specs=pl.BlockSpec(
            (TILE, F), lambda f, t, te, act: (jnp.minimum(t, act[0] - 1), f)),
    )
    h_s = pl.pallas_call(
        _up_body,
        grid_spec=grid_spec_up,
        out_shape=jax.ShapeDtypeStruct((P, half), jnp.float32),
        compiler_params=pltpu.CompilerParams(
            dimension_semantics=("arbitrary", "arbitrary")),
    )(tile_expert, act, x_s, w_up, w_up)

    ws_r = w_sorted.reshape(NTT, TILE, 1)
    grid_spec_dn = pltpu.PrefetchScalarGridSpec(
        num_scalar_prefetch=2,
        grid=(NTT,),
        in_specs=[
            pl.BlockSpec(
                (TILE, half),
                lambda t, te, act: (jnp.minimum(t, act[0] - 1), 0)),
            pl.BlockSpec(
                (1, D, half),
                lambda t, te, act: (te[jnp.minimum(t, act[0] - 1)], 0, 0)),
            pl.BlockSpec(
                (1, TILE, 1),
                lambda t, te, act: (jnp.minimum(t, act[0] - 1), 0, 0)),
        ],
        out_specs=pl.BlockSpec(
            (TILE, D), lambda t, te, act: (jnp.minimum(t, act[0] - 1), 0)),
    )
    out_s = pl.pallas_call(
        _down_body,
        grid_spec=grid_spec_dn,
        out_shape=jax.ShapeDtypeStruct((P, D), jnp.float32),
        compiler_params=pltpu.CompilerParams(
            dimension_semantics=("arbitrary",)),
    )(tile_expert, act, h_s, w_down, ws_r)

    rpair = _row_gather(out_s, inv, 2 * N).reshape(N, 2 * D)

    out = pl.pallas_call(
        _add_body,
        grid=(NT,),
        in_specs=[
            pl.BlockSpec((TB, 2 * D), lambda t: (t, 0)),
            pl.BlockSpec((TB, D), lambda t: (t, 0)),
        ],
        out_specs=pl.BlockSpec((TB, D), lambda t: (t, 0)),
        out_shape=jax.ShapeDtypeStruct((N, D), jnp.float32),
        compiler_params=pltpu.CompilerParams(
            dimension_semantics=("parallel",)),
    )(rpair, shared)

    return out.reshape(B, T, D)
